# Initial kernel scaffold; baseline (speedup 1.0000x reference)
import jax
import jax.numpy as jnp
from jax.experimental import pallas as pl


def kernel(input, boxes, im_scale, pdl, pdt, edge_ind, Wd, bd, Wb, bb, Wc, bc, Wgr, bgr, Wgc, bgc, Wrc, brc, Wcc, bcc):
    raise NotImplementedError("write your pallas kernel here")


# speedup vs baseline: 437.1267x; 437.1267x over previous
import jax
import jax.numpy as jnp
from jax.experimental import pallas as pl
from jax.experimental.pallas import tpu as pltpu

_IMG_H = 1024.0
_IMG_W = 1024.0
_ALPHA = 5.0


def _conv_body(x9_ref, w9_ref, b_ref, out_ref):
    acc = jnp.dot(x9_ref[...], w9_ref[...], preferred_element_type=jnp.float32)
    out_ref[...] = jnp.maximum(acc + b_ref[...], 0.0)


def _feature_body(boxes_ref, feat_ref, wc_ref, bc_ref, wb_ref, bb_ref, out_ref):
    boxes = boxes_ref[...]
    n = boxes.shape[0]
    x1 = boxes[:, 0:1]
    y1 = boxes[:, 1:2]
    x2 = boxes[:, 2:3]
    y2 = boxes[:, 3:4]

    bf0 = (x2 + x1) / 2.0 / _IMG_W
    bf1 = (y2 + y1) / 2.0 / _IMG_H
    bf2 = (x2 - x1) / _IMG_W
    bf3 = (y2 - y1) / _IMG_H
    bfe = (bf0 * wb_ref[0:1, :] + bf1 * wb_ref[1:2, :]
           + bf2 * wb_ref[2:3, :] + bf3 * wb_ref[3:4, :])
    box_feat = jnp.maximum(bfe + bb_ref[...], 0.0)

    hw = feat_ref.shape[0]
    size = 32
    roi_w = jnp.maximum(x2 - x1, 1.0)
    roi_h = jnp.maximum(y2 - y1, 1.0)
    rw4 = roi_w / 4.0
    rh4 = roi_h / 4.0
    u = jax.lax.broadcasted_iota(jnp.int32, (n, hw), 1)
    ybig = u // size
    xbig = u % size

    def axis_maps(base, r4, big):
        outs = []
        for py in range(2):
            acc = None
            for s in (2 * py, 2 * py + 1):
                pos = base + (s + 0.5) * r4
                valid = (pos >= -1.0) & (pos <= float(size))
                pc = jnp.clip(pos, 0.0, size - 1.0)
                p0 = jnp.floor(pc)
                p0i = p0.astype(jnp.int32)
                p1i = jnp.minimum(p0i + 1, size - 1)
                lw = pc - p0
                hwt = 1.0 - lw
                vf = jnp.where(valid, 1.0, 0.0)
                c = vf * (jnp.where(big == p0i, hwt, 0.0)
                          + jnp.where(big == p1i, lw, 0.0))
                acc = c if acc is None else acc + c
            outs.append(acc * 0.5)
        return outs

    bys = axis_maps(y1, rh4, ybig)
    bxs = axis_maps(x1, rw4, xbig)
    featv = feat_ref[...]
    acc2 = jnp.zeros((n, wc_ref.shape[2]), jnp.float32)
    for p in range(4):
        kp = bys[p // 2] * bxs[p % 2]
        pooled = jnp.dot(kp, featv, preferred_element_type=jnp.float32)
        acc2 = acc2 + jnp.dot(pooled, wc_ref[p],
                              preferred_element_type=jnp.float32)
    cnn_feat = jnp.maximum(acc2 + bc_ref[...], 0.0)

    be = box_feat.shape[1]
    out_ref[:, 0:be] = box_feat
    out_ref[:, be:] = cnn_feat


def _make_gcn_body(use_x):
    c_lo, c_hi = (0, 2) if use_x else (1, 3)

    def body(boxes_ref, boxest_ref, sc_ref, pdl_ref, pdt_ref, fus_ref,
             wg_ref, bg_ref, wh_ref, bh_ref, out_ref, scr):
        n = boxes_ref.shape[0]
        kf = 8.0 * n
        ims = sc_ref[...]
        pad = pdl_ref[...] if use_x else pdt_ref[...]
        b = boxes_ref[...]
        bt = boxest_ref[...]

        o1 = (b[:, c_lo:c_lo + 1] - pad) / ims
        o2 = (b[:, c_hi:c_hi + 1] - pad) / ims
        r1 = (bt[c_lo:c_lo + 1, :] - pad) / ims
        r2 = (bt[c_hi:c_hi + 1, :] - pad) / ims
        oc_c = (o1 + o2) / 2.0
        oc_r = (r1 + r2) / 2.0
        tb = jnp.maximum(jnp.max(o1, keepdims=True),
                         jnp.max(o2, keepdims=True))

        ch = 256
        for c0 in range(0, n, ch):
            iic = jax.lax.broadcasted_iota(jnp.int32, (ch, n), 0) + c0
            jjc = jax.lax.broadcasted_iota(jnp.int32, (ch, n), 1)
            d = (oc_c[c0:c0 + ch, :] - oc_r) * _ALPHA / tb
            scr[c0:c0 + ch, :] = jnp.where(iic == jjc, -1.0,
                                           jnp.exp(-jnp.square(d)))

        def count_ge(t):
            tot = jnp.zeros((1, 1), jnp.float32)
            for c0 in range(0, n, ch):
                tot = tot + jnp.sum(
                    jnp.where(scr[c0:c0 + ch, :] >= t, 1.0, 0.0),
                    keepdims=True)
            return tot

        def bis(_, carry):
            lo, hi = carry
            mid = (lo + hi) // 2
            t = jax.lax.bitcast_convert_type(mid, jnp.float32)
            pred = count_ge(t) >= kf
            return jnp.where(pred, mid, lo), jnp.where(pred, hi, mid)

        lo0 = jnp.zeros((1, 1), jnp.int32)
        hi0 = jnp.full((1, 1), 0x3F800001, jnp.int32)
        lo, _ = jax.lax.fori_loop(0, 31, bis, (lo0, hi0))
        tstar = jax.lax.bitcast_convert_type(lo, jnp.float32)

        cnt_gt = jnp.zeros((1, 1), jnp.float32)
        for c0 in range(0, n, ch):
            cnt_gt = cnt_gt + jnp.sum(
                jnp.where(scr[c0:c0 + ch, :] > tstar, 1.0, 0.0),
                keepdims=True)
        m = kf - cnt_gt

        def bis2(_, carry):
            lo2, hi2 = carry
            mid = (lo2 + hi2) // 2
            cnt = jnp.zeros((1, 1), jnp.float32)
            for c0 in range(0, n, ch):
                iic = jax.lax.broadcasted_iota(jnp.int32, (ch, n), 0) + c0
                jjc = jax.lax.broadcasted_iota(jnp.int32, (ch, n), 1)
                cnt = cnt + jnp.sum(
                    jnp.where((scr[c0:c0 + ch, :] == tstar)
                              & ((iic * n + jjc) <= mid), 1.0, 0.0),
                    keepdims=True)
            pred = cnt <= m
            return jnp.where(pred, mid, lo2), jnp.where(pred, hi2, mid)

        lo20 = jnp.full((1, 1), -1, jnp.int32)
        hi20 = jnp.full((1, 1), n * n, jnp.int32)
        lstar, _ = jax.lax.fori_loop(0, 24, bis2, (lo20, hi20))

        for c0 in range(0, n, ch):
            iic = jax.lax.broadcasted_iota(jnp.int32, (ch, n), 0) + c0
            jjc = jax.lax.broadcasted_iota(jnp.int32, (ch, n), 1)
            av = scr[c0:c0 + ch, :]
            keep = (av > tstar) | ((av == tstar) & ((jjc * n + iic) <= lstar))
            a = jnp.where(keep, av, 0.0)
            scr[c0:c0 + ch, :] = jnp.where(iic == jjc, 1.0, a)
        degs = [jnp.sum(scr[c0:c0 + ch, :], axis=1, keepdims=True)
                for c0 in range(0, n, ch)]
        deg = jnp.concatenate(degs, axis=0)
        dis = jnp.where(deg > 0,
                        jax.lax.rsqrt(jnp.maximum(deg, 1e-12)), 0.0)
        xw = jnp.dot(fus_ref[...], wg_ref[...],
                     preferred_element_type=jnp.float32)
        xws = dis * xw
        for c0 in range(0, n, ch):
            g = jnp.dot(scr[c0:c0 + ch, :], xws,
                        preferred_element_type=jnp.float32)
            gf = jnp.maximum(dis[c0:c0 + ch, :] * g + bg_ref[...], 0.0)
            head = jnp.dot(gf, wh_ref[...],
                           preferred_element_type=jnp.float32) + bh_ref[...]
            out_ref[c0:c0 + ch, :] = jnp.where(head >= 0.0, head, 0.01 * head)

    return body


def kernel(input, boxes, im_scale, pdl, pdt, edge_ind,
           Wd, bd, Wb, bb, Wc, bc, Wgr, bgr, Wgc, bgc,
           Wrc, brc, Wcc, bcc):
    n = boxes.shape[0]
    cin = input.shape[1]
    h = input.shape[2]
    w = input.shape[3]
    cout = Wd.shape[0]

    xp = jnp.pad(input[0], ((0, 0), (1, 1), (1, 1)))
    x9 = jnp.concatenate(
        [xp[:, dy:dy + h, dx:dx + w] for dy in range(3) for dx in range(3)],
        axis=0).reshape(9 * cin, h * w).T
    w9 = jnp.transpose(Wd, (2, 3, 1, 0)).reshape(9 * cin, cout)
    feat_t = pl.pallas_call(
        _conv_body,
        out_shape=jax.ShapeDtypeStruct((h * w, cout), jnp.float32),
    )(x9, w9, bd.reshape(1, -1))

    wc_stack = jnp.transpose(Wc.reshape(cout, 4, Wc.shape[1]), (1, 0, 2))
    box_emb = Wb.shape[1]
    fdim = box_emb + Wc.shape[1]
    nb = 512
    fusion = pl.pallas_call(
        _feature_body,
        grid=(n // nb,),
        in_specs=[
            pl.BlockSpec((nb, 4), lambda i: (i, 0)),
            pl.BlockSpec((h * w, cout), lambda i: (0, 0)),
            pl.BlockSpec((4, cout, Wc.shape[1]), lambda i: (0, 0, 0)),
            pl.BlockSpec((1, Wc.shape[1]), lambda i: (0, 0)),
            pl.BlockSpec((4, box_emb), lambda i: (0, 0)),
            pl.BlockSpec((1, box_emb), lambda i: (0, 0)),
        ],
        out_specs=pl.BlockSpec((nb, fdim), lambda i: (i, 0)),
        out_shape=jax.ShapeDtypeStruct((n, fdim), jnp.float32),
    )(boxes, feat_t, wc_stack, bc.reshape(1, -1), Wb, bb.reshape(1, -1))

    hpad = (-Wrc.shape[1]) % 128
    wrcp = jnp.pad(Wrc, ((0, 0), (0, hpad)))
    wccp = jnp.pad(Wcc, ((0, 0), (0, hpad)))
    brcp = jnp.pad(brc, (0, hpad)).reshape(1, -1)
    bccp = jnp.pad(bcc, (0, hpad)).reshape(1, -1)
    common = (boxes, boxes.T, im_scale.reshape(1, 1), pdl.reshape(1, 1),
              pdt.reshape(1, 1), fusion)
    row_o = pl.pallas_call(
        _make_gcn_body(False),
        out_shape=jax.ShapeDtypeStruct((n, Wrc.shape[1] + hpad), jnp.float32),
        scratch_shapes=[pltpu.VMEM((n, n), jnp.float32)],
    )(*common, Wgr, bgr.reshape(1, -1), wrcp, brcp)
    col_o = pl.pallas_call(
        _make_gcn_body(True),
        out_shape=jax.ShapeDtypeStruct((n, Wcc.shape[1] + hpad), jnp.float32),
        scratch_shapes=[pltpu.VMEM((n, n), jnp.float32)],
    )(*common, Wgc, bgc.reshape(1, -1), wccp, bccp)

    nres = Wrc.shape[1] // 4
    cls_row = row_o[:, :Wrc.shape[1]].reshape(n, 2, nres, 2)
    cls_col = col_o[:, :Wcc.shape[1]].reshape(n, 2, nres, 2)
    return (cls_row, cls_col)
